# baseline (speedup 1.0000x reference)
import jax
import jax.numpy as jnp
from jax import lax
from jax.experimental import pallas as pl
from jax.experimental.pallas import tpu as pltpu
from jax.experimental.pallas import tpu_sc as plsc

NN = 10000
EE = 640000
DD = 128
HH = 32
NP2 = 10112
NW = 32
QW = 512
NR = 40
RPP = 5
NPH = 8
EPW = NR * QW
EPAD = EPW * NW
ACC_ROWS = 3 * NP2
STRIPE = 1896
ZROWS = 474
ZR2 = 79
OUTW = 632

_f32 = jnp.float32
_i32 = jnp.int32


def _mdst_body(d_ref, m1_ref, m2_ref, o1_ref, o2_ref):
    d = d_ref[...]
    o1_ref[...] = jnp.where(m1_ref[...] > 0, d + NP2, NN + NP2)
    o2_ref[...] = jnp.where(m2_ref[...] > 0, d + 2 * NP2, NN + 2 * NP2)


def _tc_mdst(dst_p, m1_p, m2_p):
    spec = pl.BlockSpec((1, 1, EPW), lambda i: (i, 0, 0))
    return pl.pallas_call(
        _mdst_body,
        grid=(NW,),
        in_specs=[spec, spec, spec],
        out_specs=[spec, spec],
        out_shape=[jax.ShapeDtypeStruct((NW, 1, EPW), _i32)] * 2,
    )(dst_p, m1_p, m2_p)


def _deg_body(md0_hbm, md1_hbm, md2_hbm, out_hbm,
              md0, md1, md2, ones, zbuf, acc, sem):
    c = lax.axis_index("c")
    s = lax.axis_index("s")
    w = c * 16 + s

    zero16 = jnp.zeros((16,), _f32)
    one_row = jnp.where(lax.iota(_i32, 16) == 0, 1.0, 0.0).astype(_f32)

    def zinit(i, carry):
        zbuf[i, :] = zero16
        return carry
    lax.fori_loop(0, ZROWS, zinit, 0)

    def oinit(i, carry):
        ones[i, :] = one_row
        return carry
    lax.fori_loop(0, QW, oinit, 0)

    for i in range(4):
        pltpu.sync_copy(zbuf, acc.at[pl.ds(s * STRIPE + i * ZROWS, ZROWS)])
    pltpu.sync_copy(md0_hbm.at[w], md0)
    pltpu.sync_copy(md1_hbm.at[w], md1)
    pltpu.sync_copy(md2_hbm.at[w], md2)
    plsc.subcore_barrier()

    def chunk(j, carry):
        s0 = pltpu.async_copy(ones, acc.at[md0.at[j]], sem, add=True)
        s1 = pltpu.async_copy(ones, acc.at[md1.at[j]], sem, add=True)
        s2 = pltpu.async_copy(ones, acc.at[md2.at[j]], sem, add=True)
        s0.wait()
        s1.wait()
        s2.wait()
        return carry
    lax.fori_loop(0, NR, chunk, 0)
    plsc.subcore_barrier()

    for v in range(3):
        pltpu.sync_copy(acc.at[pl.ds(v * NP2 + s * OUTW, OUTW)],
                        out_hbm.at[c, v, pl.ds(s * OUTW, OUTW)])


def _sc_deg(md0_r, md1_r, md2_r):
    mesh = plsc.VectorSubcoreMesh(core_axis_name="c", subcore_axis_name="s")
    return pl.kernel(
        _deg_body,
        out_type=jax.ShapeDtypeStruct((2, 3, NP2, 16), _f32),
        mesh=mesh,
        compiler_params=pltpu.CompilerParams(use_tc_tiling_on_sc=False),
        scratch_types=[
            pltpu.VMEM((NR, QW), _i32),
            pltpu.VMEM((NR, QW), _i32),
            pltpu.VMEM((NR, QW), _i32),
            pltpu.VMEM((QW, 16), _f32),
            pltpu.VMEM((ZROWS, 16), _f32),
            pltpu.VMEM_SHARED((ACC_ROWS, 16), _f32),
            pltpu.SemaphoreType.DMA,
        ],
    )(md0_r, md1_r, md2_r)


def _dense1_body(deg_ref, x_ref, w1_ref, f1_ref, f2_ref, g_ref, db_ref):
    x = x_ref[...]
    w1 = w1_ref[...]
    f1 = f1_ref[...]
    f2 = f2_ref[...]
    ws = (w1, w1 * f1.reshape(DD, 1), w1 * f2.reshape(DD, 1))
    for v in range(3):
        dsum = deg_ref[v] + deg_ref[v + 3]
        dinv = lax.rsqrt(1.0 + dsum[:, 0:1])
        h = jnp.dot(x, ws[v], preferred_element_type=_f32)
        g_ref[v] = dinv * h
        db_ref[v] = jnp.broadcast_to(dinv, h.shape)


def _tc_dense1(deg6, x, W1, fm1, fm2):
    nb = 10
    blk = NN // nb
    return pl.pallas_call(
        _dense1_body,
        grid=(nb,),
        in_specs=[
            pl.BlockSpec((6, blk, 16), lambda i: (0, i, 0)),
            pl.BlockSpec((blk, DD), lambda i: (i, 0)),
            pl.BlockSpec((DD, HH), lambda i: (0, 0)),
            pl.BlockSpec((1, DD), lambda i: (0, 0)),
            pl.BlockSpec((1, DD), lambda i: (0, 0)),
        ],
        out_specs=[
            pl.BlockSpec((3, blk, HH), lambda i: (0, i, 0)),
            pl.BlockSpec((3, blk, HH), lambda i: (0, i, 0)),
        ],
        out_shape=[jax.ShapeDtypeStruct((3, NN, HH), _f32)] * 2,
    )(deg6, x, W1, fm1, fm2)


def _spmm_body(g0_hbm, g1_hbm, g2_hbm, src_hbm, md0_hbm, md1_hbm, md2_hbm,
               out_hbm, srcv, md0, md1, md2, buf0, buf1, buf2, zbuf, acc, sem, sem2):
    c = lax.axis_index("c")
    s = lax.axis_index("s")
    w = c * 16 + s

    zero16 = jnp.zeros((16,), _f32)

    def zinit(i, carry):
        zbuf[i, pl.ds(0, 16)] = zero16
        zbuf[i, pl.ds(16, 16)] = zero16
        return carry
    lax.fori_loop(0, ZR2, zinit, 0)

    for i in range(24):
        pltpu.sync_copy(zbuf, acc.at[pl.ds(s * STRIPE + i * ZR2, ZR2)])
    plsc.subcore_barrier()

    def chunk(j, carry):
        cp0 = pltpu.async_copy(g0_hbm.at[srcv.at[j]], buf0, sem)
        cp1 = pltpu.async_copy(g1_hbm.at[srcv.at[j]], buf1, sem)
        cp2 = pltpu.async_copy(g2_hbm.at[srcv.at[j]], buf2, sem)
        cp0.wait()
        cp1.wait()
        cp2.wait()
        s0 = pltpu.async_copy(buf0, acc.at[md0.at[j]], sem2, add=True)
        s1 = pltpu.async_copy(buf1, acc.at[md1.at[j]], sem2, add=True)
        s2 = pltpu.async_copy(buf2, acc.at[md2.at[j]], sem2, add=True)
        s0.wait()
        s1.wait()
        s2.wait()
        return carry

    for p in range(NPH):
        pltpu.sync_copy(src_hbm.at[w, pl.ds(p * RPP, RPP)], srcv)
        pltpu.sync_copy(md0_hbm.at[w, pl.ds(p * RPP, RPP)], md0)
        pltpu.sync_copy(md1_hbm.at[w, pl.ds(p * RPP, RPP)], md1)
        pltpu.sync_copy(md2_hbm.at[w, pl.ds(p * RPP, RPP)], md2)
        lax.fori_loop(0, RPP, chunk, 0)
    plsc.subcore_barrier()

    for v in range(3):
        pltpu.sync_copy(acc.at[pl.ds(v * NP2 + s * OUTW, OUTW)],
                        out_hbm.at[c, v, pl.ds(s * OUTW, OUTW)])


def _sc_spmm(g3, src_r, md0_r, md1_r, md2_r):
    mesh = plsc.VectorSubcoreMesh(core_axis_name="c", subcore_axis_name="s")
    return pl.kernel(
        _spmm_body,
        out_type=jax.ShapeDtypeStruct((2, 3, NP2, HH), _f32),
        mesh=mesh,
        compiler_params=pltpu.CompilerParams(use_tc_tiling_on_sc=False),
        scratch_types=[
            pltpu.VMEM((RPP, QW), _i32),
            pltpu.VMEM((RPP, QW), _i32),
            pltpu.VMEM((RPP, QW), _i32),
            pltpu.VMEM((RPP, QW), _i32),
            pltpu.VMEM((QW, HH), _f32),
            pltpu.VMEM((QW, HH), _f32),
            pltpu.VMEM((QW, HH), _f32),
            pltpu.VMEM((ZR2, HH), _f32),
            pltpu.VMEM_SHARED((ACC_ROWS, HH), _f32),
            pltpu.SemaphoreType.DMA,
            pltpu.SemaphoreType.DMA,
        ],
    )(g3[0], g3[1], g3[2], src_r, md0_r, md1_r, md2_r)


def _mid_body(acc_ref, g_ref, db_ref, b1_ref, w2_ref, g2_ref):
    w2 = w2_ref[...]
    b1 = b1_ref[...]
    for v in range(3):
        db = db_ref[v]
        t = db * (acc_ref[v] + acc_ref[v + 3] + g_ref[v]) + b1
        z = jnp.maximum(t, 0.0)
        g2_ref[v] = db * jnp.dot(z, w2, preferred_element_type=_f32)


def _tc_mid(acc6, G1, DB, b1, W2):
    nb = 10
    blk = NN // nb
    return pl.pallas_call(
        _mid_body,
        grid=(nb,),
        in_specs=[
            pl.BlockSpec((6, blk, HH), lambda i: (0, i, 0)),
            pl.BlockSpec((3, blk, HH), lambda i: (0, i, 0)),
            pl.BlockSpec((3, blk, HH), lambda i: (0, i, 0)),
            pl.BlockSpec((1, HH), lambda i: (0, 0)),
            pl.BlockSpec((HH, HH), lambda i: (0, 0)),
        ],
        out_specs=pl.BlockSpec((3, blk, HH), lambda i: (0, i, 0)),
        out_shape=jax.ShapeDtypeStruct((3, NN, HH), _f32),
    )(acc6, G1, DB, b1, W2)


def _fin_body(acc_ref, g_ref, db_ref, b2_ref, z_ref):
    b2 = b2_ref[...]
    for v in range(3):
        db = db_ref[v]
        t = db * (acc_ref[v] + acc_ref[v + 3] + g_ref[v]) + b2
        z_ref[v] = jnp.maximum(t, 0.0)


def _tc_fin(acc6, G2, DB, b2):
    nb = 10
    blk = NN // nb
    return pl.pallas_call(
        _fin_body,
        grid=(nb,),
        in_specs=[
            pl.BlockSpec((6, blk, HH), lambda i: (0, i, 0)),
            pl.BlockSpec((3, blk, HH), lambda i: (0, i, 0)),
            pl.BlockSpec((3, blk, HH), lambda i: (0, i, 0)),
            pl.BlockSpec((1, HH), lambda i: (0, 0)),
        ],
        out_specs=pl.BlockSpec((3, blk, HH), lambda i: (0, i, 0)),
        out_shape=jax.ShapeDtypeStruct((3, NN, HH), _f32),
    )(acc6, G2, DB, b2)


def kernel(x, edge_index, W1, b1, W2, b2,
           edge_mask1, feat_mask1, edge_mask2, feat_mask2):
    src = edge_index[0]
    dst = edge_index[1]
    npad = EPAD - EE
    src_p = jnp.concatenate([src, jnp.zeros((npad,), _i32)])
    dst_p = jnp.concatenate([dst, jnp.full((npad,), NN, _i32)])
    m1_p = jnp.concatenate([edge_mask1.astype(_i32), jnp.zeros((npad,), _i32)])
    m2_p = jnp.concatenate([edge_mask2.astype(_i32), jnp.zeros((npad,), _i32)])

    mdst1, mdst2 = _tc_mdst(dst_p.reshape(NW, 1, EPW),
                            m1_p.reshape(NW, 1, EPW),
                            m2_p.reshape(NW, 1, EPW))

    src_r = src_p.reshape(NW, NR, QW)
    md0_r = dst_p.reshape(NW, NR, QW)
    md1_r = mdst1.reshape(NW, NR, QW)
    md2_r = mdst2.reshape(NW, NR, QW)

    deg = _sc_deg(md0_r, md1_r, md2_r)
    deg6 = deg[:, :, :NN, :].reshape(6, NN, 16)

    fm1 = feat_mask1.astype(_f32).reshape(1, DD)
    fm2 = feat_mask2.astype(_f32).reshape(1, DD)
    G1, DB = _tc_dense1(deg6, x, W1, fm1, fm2)

    acc1 = _sc_spmm(G1, src_r, md0_r, md1_r, md2_r)[:, :, :NN, :].reshape(6, NN, HH)
    G2 = _tc_mid(acc1, G1, DB, b1.reshape(1, HH), W2)
    acc2 = _sc_spmm(G2, src_r, md0_r, md1_r, md2_r)[:, :, :NN, :].reshape(6, NN, HH)
    Z = _tc_fin(acc2, G2, DB, b2.reshape(1, HH))
    return (Z[0], Z[1], Z[2])


# speedup vs baseline: 1.0093x; 1.0093x over previous
import jax
import jax.numpy as jnp
from jax import lax
from jax.experimental import pallas as pl
from jax.experimental.pallas import tpu as pltpu
from jax.experimental.pallas import tpu_sc as plsc

NN = 10000
EE = 640000
DD = 128
HH = 32
NP2 = 10112
NW = 32
QW = 256
NR = 80
RPP = 10
NPH = 8
EPW = NR * QW
EPAD = EPW * NW
ACC_ROWS = 3 * NP2
STRIPE = 1896
ZROWS = 474
ZR2 = 79
OUTW = 632

_f32 = jnp.float32
_i32 = jnp.int32


def _mdst_body(d_ref, m1_ref, m2_ref, o1_ref, o2_ref):
    d = d_ref[...]
    o1_ref[...] = jnp.where(m1_ref[...] > 0, d + NP2, NN + NP2)
    o2_ref[...] = jnp.where(m2_ref[...] > 0, d + 2 * NP2, NN + 2 * NP2)


def _tc_mdst(dst_p, m1_p, m2_p):
    spec = pl.BlockSpec((1, 1, EPW), lambda i: (i, 0, 0))
    return pl.pallas_call(
        _mdst_body,
        grid=(NW,),
        in_specs=[spec, spec, spec],
        out_specs=[spec, spec],
        out_shape=[jax.ShapeDtypeStruct((NW, 1, EPW), _i32)] * 2,
    )(dst_p, m1_p, m2_p)


def _deg_body(md0_hbm, md1_hbm, md2_hbm, out_hbm,
              md0, md1, md2, ones, zbuf, acc, sem):
    c = lax.axis_index("c")
    s = lax.axis_index("s")
    w = c * 16 + s

    zero16 = jnp.zeros((16,), _f32)
    one_row = jnp.where(lax.iota(_i32, 16) == 0, 1.0, 0.0).astype(_f32)

    def zinit(i, carry):
        zbuf[i, :] = zero16
        return carry
    lax.fori_loop(0, ZROWS, zinit, 0)

    def oinit(i, carry):
        ones[i, :] = one_row
        return carry
    lax.fori_loop(0, QW, oinit, 0)

    for i in range(4):
        pltpu.sync_copy(zbuf, acc.at[pl.ds(s * STRIPE + i * ZROWS, ZROWS)])
    pltpu.sync_copy(md0_hbm.at[w], md0)
    pltpu.sync_copy(md1_hbm.at[w], md1)
    pltpu.sync_copy(md2_hbm.at[w], md2)
    plsc.subcore_barrier()

    def chunk(j, carry):
        s0 = pltpu.async_copy(ones, acc.at[md0.at[j]], sem, add=True)
        s1 = pltpu.async_copy(ones, acc.at[md1.at[j]], sem, add=True)
        s2 = pltpu.async_copy(ones, acc.at[md2.at[j]], sem, add=True)
        s0.wait()
        s1.wait()
        s2.wait()
        return carry
    lax.fori_loop(0, NR, chunk, 0)
    plsc.subcore_barrier()

    for v in range(3):
        pltpu.sync_copy(acc.at[pl.ds(v * NP2 + s * OUTW, OUTW)],
                        out_hbm.at[c, v, pl.ds(s * OUTW, OUTW)])


def _sc_deg(md0_r, md1_r, md2_r):
    mesh = plsc.VectorSubcoreMesh(core_axis_name="c", subcore_axis_name="s")
    return pl.kernel(
        _deg_body,
        out_type=jax.ShapeDtypeStruct((2, 3, NP2, 16), _f32),
        mesh=mesh,
        compiler_params=pltpu.CompilerParams(use_tc_tiling_on_sc=False),
        scratch_types=[
            pltpu.VMEM((NR, QW), _i32),
            pltpu.VMEM((NR, QW), _i32),
            pltpu.VMEM((NR, QW), _i32),
            pltpu.VMEM((QW, 16), _f32),
            pltpu.VMEM((ZROWS, 16), _f32),
            pltpu.VMEM_SHARED((ACC_ROWS, 16), _f32),
            pltpu.SemaphoreType.DMA,
        ],
    )(md0_r, md1_r, md2_r)


def _dense1_body(deg_ref, x_ref, w1_ref, f1_ref, f2_ref, g_ref, db_ref):
    x = x_ref[...]
    w1 = w1_ref[...]
    f1 = f1_ref[...]
    f2 = f2_ref[...]
    ws = (w1, w1 * f1.reshape(DD, 1), w1 * f2.reshape(DD, 1))
    for v in range(3):
        dsum = deg_ref[v] + deg_ref[v + 3]
        dinv = lax.rsqrt(1.0 + dsum[:, 0:1])
        h = jnp.dot(x, ws[v], preferred_element_type=_f32)
        g_ref[v] = dinv * h
        db_ref[v] = jnp.broadcast_to(dinv, h.shape)


def _tc_dense1(deg6, x, W1, fm1, fm2):
    nb = 10
    blk = NN // nb
    return pl.pallas_call(
        _dense1_body,
        grid=(nb,),
        in_specs=[
            pl.BlockSpec((6, blk, 16), lambda i: (0, i, 0)),
            pl.BlockSpec((blk, DD), lambda i: (i, 0)),
            pl.BlockSpec((DD, HH), lambda i: (0, 0)),
            pl.BlockSpec((1, DD), lambda i: (0, 0)),
            pl.BlockSpec((1, DD), lambda i: (0, 0)),
        ],
        out_specs=[
            pl.BlockSpec((3, blk, HH), lambda i: (0, i, 0)),
            pl.BlockSpec((3, blk, HH), lambda i: (0, i, 0)),
        ],
        out_shape=[jax.ShapeDtypeStruct((3, NN, HH), _f32)] * 2,
    )(deg6, x, W1, fm1, fm2)


def _spmm_body(g0_hbm, g1_hbm, g2_hbm, src_hbm, md0_hbm, md1_hbm, md2_hbm,
               out_hbm, srcv, md0, md1, md2, bA0, bA1, bA2, bB0, bB1, bB2,
               zbuf, acc, semgA, semgB, semsA, semsB):
    c = lax.axis_index("c")
    s = lax.axis_index("s")
    w = c * 16 + s

    zero16 = jnp.zeros((16,), _f32)

    def zinit(i, carry):
        zbuf[i, pl.ds(0, 16)] = zero16
        zbuf[i, pl.ds(16, 16)] = zero16
        return carry
    lax.fori_loop(0, ZR2, zinit, 0)

    for i in range(24):
        pltpu.sync_copy(zbuf, acc.at[pl.ds(s * STRIPE + i * ZR2, ZR2)])
    plsc.subcore_barrier()

    def pair(k, carry):
        jA = 2 * k
        jB = 2 * k + 1
        cA0 = pltpu.async_copy(g0_hbm.at[srcv.at[jA]], bA0, semgA)
        cA1 = pltpu.async_copy(g1_hbm.at[srcv.at[jA]], bA1, semgA)
        cA2 = pltpu.async_copy(g2_hbm.at[srcv.at[jA]], bA2, semgA)
        cB0 = pltpu.async_copy(g0_hbm.at[srcv.at[jB]], bB0, semgB)
        cB1 = pltpu.async_copy(g1_hbm.at[srcv.at[jB]], bB1, semgB)
        cB2 = pltpu.async_copy(g2_hbm.at[srcv.at[jB]], bB2, semgB)
        cA0.wait()
        cA1.wait()
        cA2.wait()
        sA0 = pltpu.async_copy(bA0, acc.at[md0.at[jA]], semsA, add=True)
        sA1 = pltpu.async_copy(bA1, acc.at[md1.at[jA]], semsA, add=True)
        sA2 = pltpu.async_copy(bA2, acc.at[md2.at[jA]], semsA, add=True)
        cB0.wait()
        cB1.wait()
        cB2.wait()
        sB0 = pltpu.async_copy(bB0, acc.at[md0.at[jB]], semsB, add=True)
        sB1 = pltpu.async_copy(bB1, acc.at[md1.at[jB]], semsB, add=True)
        sB2 = pltpu.async_copy(bB2, acc.at[md2.at[jB]], semsB, add=True)
        sA0.wait()
        sA1.wait()
        sA2.wait()
        sB0.wait()
        sB1.wait()
        sB2.wait()
        return carry

    for p in range(NPH):
        pltpu.sync_copy(src_hbm.at[w, pl.ds(p * RPP, RPP)], srcv)
        pltpu.sync_copy(md0_hbm.at[w, pl.ds(p * RPP, RPP)], md0)
        pltpu.sync_copy(md1_hbm.at[w, pl.ds(p * RPP, RPP)], md1)
        pltpu.sync_copy(md2_hbm.at[w, pl.ds(p * RPP, RPP)], md2)
        lax.fori_loop(0, RPP // 2, pair, 0)
    plsc.subcore_barrier()

    for v in range(3):
        pltpu.sync_copy(acc.at[pl.ds(v * NP2 + s * OUTW, OUTW)],
                        out_hbm.at[c, v, pl.ds(s * OUTW, OUTW)])


def _sc_spmm(g3, src_r, md0_r, md1_r, md2_r):
    mesh = plsc.VectorSubcoreMesh(core_axis_name="c", subcore_axis_name="s")
    return pl.kernel(
        _spmm_body,
        out_type=jax.ShapeDtypeStruct((2, 3, NP2, HH), _f32),
        mesh=mesh,
        compiler_params=pltpu.CompilerParams(use_tc_tiling_on_sc=False),
        scratch_types=[
            pltpu.VMEM((RPP, QW), _i32),
            pltpu.VMEM((RPP, QW), _i32),
            pltpu.VMEM((RPP, QW), _i32),
            pltpu.VMEM((RPP, QW), _i32),
            pltpu.VMEM((QW, HH), _f32),
            pltpu.VMEM((QW, HH), _f32),
            pltpu.VMEM((QW, HH), _f32),
            pltpu.VMEM((QW, HH), _f32),
            pltpu.VMEM((QW, HH), _f32),
            pltpu.VMEM((QW, HH), _f32),
            pltpu.VMEM((ZR2, HH), _f32),
            pltpu.VMEM_SHARED((ACC_ROWS, HH), _f32),
            pltpu.SemaphoreType.DMA,
            pltpu.SemaphoreType.DMA,
            pltpu.SemaphoreType.DMA,
            pltpu.SemaphoreType.DMA,
        ],
    )(g3[0], g3[1], g3[2], src_r, md0_r, md1_r, md2_r)


def _mid_body(acc_ref, g_ref, db_ref, b1_ref, w2_ref, g2_ref):
    w2 = w2_ref[...]
    b1 = b1_ref[...]
    for v in range(3):
        db = db_ref[v]
        t = db * (acc_ref[v] + acc_ref[v + 3] + g_ref[v]) + b1
        z = jnp.maximum(t, 0.0)
        g2_ref[v] = db * jnp.dot(z, w2, preferred_element_type=_f32)


def _tc_mid(acc6, G1, DB, b1, W2):
    nb = 10
    blk = NN // nb
    return pl.pallas_call(
        _mid_body,
        grid=(nb,),
        in_specs=[
            pl.BlockSpec((6, blk, HH), lambda i: (0, i, 0)),
            pl.BlockSpec((3, blk, HH), lambda i: (0, i, 0)),
            pl.BlockSpec((3, blk, HH), lambda i: (0, i, 0)),
            pl.BlockSpec((1, HH), lambda i: (0, 0)),
            pl.BlockSpec((HH, HH), lambda i: (0, 0)),
        ],
        out_specs=pl.BlockSpec((3, blk, HH), lambda i: (0, i, 0)),
        out_shape=jax.ShapeDtypeStruct((3, NN, HH), _f32),
    )(acc6, G1, DB, b1, W2)


def _fin_body(acc_ref, g_ref, db_ref, b2_ref, z_ref):
    b2 = b2_ref[...]
    for v in range(3):
        db = db_ref[v]
        t = db * (acc_ref[v] + acc_ref[v + 3] + g_ref[v]) + b2
        z_ref[v] = jnp.maximum(t, 0.0)


def _tc_fin(acc6, G2, DB, b2):
    nb = 10
    blk = NN // nb
    return pl.pallas_call(
        _fin_body,
        grid=(nb,),
        in_specs=[
            pl.BlockSpec((6, blk, HH), lambda i: (0, i, 0)),
            pl.BlockSpec((3, blk, HH), lambda i: (0, i, 0)),
            pl.BlockSpec((3, blk, HH), lambda i: (0, i, 0)),
            pl.BlockSpec((1, HH), lambda i: (0, 0)),
        ],
        out_specs=pl.BlockSpec((3, blk, HH), lambda i: (0, i, 0)),
        out_shape=jax.ShapeDtypeStruct((3, NN, HH), _f32),
    )(acc6, G2, DB, b2)


def kernel(x, edge_index, W1, b1, W2, b2,
           edge_mask1, feat_mask1, edge_mask2, feat_mask2):
    src = edge_index[0]
    dst = edge_index[1]
    npad = EPAD - EE
    src_p = jnp.concatenate([src, jnp.zeros((npad,), _i32)])
    dst_p = jnp.concatenate([dst, jnp.full((npad,), NN, _i32)])
    m1_p = jnp.concatenate([edge_mask1.astype(_i32), jnp.zeros((npad,), _i32)])
    m2_p = jnp.concatenate([edge_mask2.astype(_i32), jnp.zeros((npad,), _i32)])

    mdst1, mdst2 = _tc_mdst(dst_p.reshape(NW, 1, EPW),
                            m1_p.reshape(NW, 1, EPW),
                            m2_p.reshape(NW, 1, EPW))

    src_r = src_p.reshape(NW, NR, QW)
    md0_r = dst_p.reshape(NW, NR, QW)
    md1_r = mdst1.reshape(NW, NR, QW)
    md2_r = mdst2.reshape(NW, NR, QW)

    deg = _sc_deg(md0_r, md1_r, md2_r)
    deg6 = deg[:, :, :NN, :].reshape(6, NN, 16)

    fm1 = feat_mask1.astype(_f32).reshape(1, DD)
    fm2 = feat_mask2.astype(_f32).reshape(1, DD)
    G1, DB = _tc_dense1(deg6, x, W1, fm1, fm2)

    acc1 = _sc_spmm(G1, src_r, md0_r, md1_r, md2_r)[:, :, :NN, :].reshape(6, NN, HH)
    G2 = _tc_mid(acc1, G1, DB, b1.reshape(1, HH), W2)
    acc2 = _sc_spmm(G2, src_r, md0_r, md1_r, md2_r)[:, :, :NN, :].reshape(6, NN, HH)
    Z = _tc_fin(acc2, G2, DB, b2.reshape(1, HH))
    return (Z[0], Z[1], Z[2])


# speedup vs baseline: 1.1050x; 1.0948x over previous
import jax
import jax.numpy as jnp
from jax import lax
from jax.experimental import pallas as pl
from jax.experimental.pallas import tpu as pltpu
from jax.experimental.pallas import tpu_sc as plsc

NN = 10000
EE = 640000
DD = 128
HH = 32
NP2 = 10112
NW = 32
QW = 128
NR = 160
RPP = 20
NPH = 8
EPW = NR * QW
EPAD = EPW * NW
ACC_ROWS = 3 * NP2
STRIPE = 1896
ZROWS = 474
ZR2 = 79
OUTW = 632

_f32 = jnp.float32
_i32 = jnp.int32


def _mdst_body(d_ref, m1_ref, m2_ref, o1_ref, o2_ref):
    d = d_ref[...]
    o1_ref[...] = jnp.where(m1_ref[...] > 0, d + NP2, NN + NP2)
    o2_ref[...] = jnp.where(m2_ref[...] > 0, d + 2 * NP2, NN + 2 * NP2)


def _tc_mdst(dst_p, m1_p, m2_p):
    spec = pl.BlockSpec((1, 1, EPW), lambda i: (i, 0, 0))
    return pl.pallas_call(
        _mdst_body,
        grid=(NW,),
        in_specs=[spec, spec, spec],
        out_specs=[spec, spec],
        out_shape=[jax.ShapeDtypeStruct((NW, 1, EPW), _i32)] * 2,
    )(dst_p, m1_p, m2_p)


def _deg_body(md0_hbm, md1_hbm, md2_hbm, out_hbm,
              md0, md1, md2, ones, zbuf, acc, sem):
    c = lax.axis_index("c")
    s = lax.axis_index("s")
    w = c * 16 + s

    zero16 = jnp.zeros((16,), _f32)
    one_row = jnp.where(lax.iota(_i32, 16) == 0, 1.0, 0.0).astype(_f32)

    def zinit(i, carry):
        zbuf[i, :] = zero16
        return carry
    lax.fori_loop(0, ZROWS, zinit, 0)

    def oinit(i, carry):
        ones[i, :] = one_row
        return carry
    lax.fori_loop(0, QW, oinit, 0)

    for i in range(4):
        pltpu.sync_copy(zbuf, acc.at[pl.ds(s * STRIPE + i * ZROWS, ZROWS)])
    pltpu.sync_copy(md0_hbm.at[w], md0)
    pltpu.sync_copy(md1_hbm.at[w], md1)
    pltpu.sync_copy(md2_hbm.at[w], md2)
    plsc.subcore_barrier()

    def chunk(j, carry):
        s0 = pltpu.async_copy(ones, acc.at[md0.at[j]], sem, add=True)
        s1 = pltpu.async_copy(ones, acc.at[md1.at[j]], sem, add=True)
        s2 = pltpu.async_copy(ones, acc.at[md2.at[j]], sem, add=True)
        s0.wait()
        s1.wait()
        s2.wait()
        return carry
    lax.fori_loop(0, NR, chunk, 0)
    plsc.subcore_barrier()

    for v in range(3):
        pltpu.sync_copy(acc.at[pl.ds(v * NP2 + s * OUTW, OUTW)],
                        out_hbm.at[c, v, pl.ds(s * OUTW, OUTW)])


def _sc_deg(md0_r, md1_r, md2_r):
    mesh = plsc.VectorSubcoreMesh(core_axis_name="c", subcore_axis_name="s")
    return pl.kernel(
        _deg_body,
        out_type=jax.ShapeDtypeStruct((2, 3, NP2, 16), _f32),
        mesh=mesh,
        compiler_params=pltpu.CompilerParams(use_tc_tiling_on_sc=False),
        scratch_types=[
            pltpu.VMEM((NR, QW), _i32),
            pltpu.VMEM((NR, QW), _i32),
            pltpu.VMEM((NR, QW), _i32),
            pltpu.VMEM((QW, 16), _f32),
            pltpu.VMEM((ZROWS, 16), _f32),
            pltpu.VMEM_SHARED((ACC_ROWS, 16), _f32),
            pltpu.SemaphoreType.DMA,
        ],
    )(md0_r, md1_r, md2_r)


def _dense1_body(deg_ref, x_ref, w1_ref, f1_ref, f2_ref, g_ref, db_ref):
    x = x_ref[...]
    w1 = w1_ref[...]
    f1 = f1_ref[...]
    f2 = f2_ref[...]
    ws = (w1, w1 * f1.reshape(DD, 1), w1 * f2.reshape(DD, 1))
    for v in range(3):
        dsum = deg_ref[v] + deg_ref[v + 3]
        dinv = lax.rsqrt(1.0 + dsum[:, 0:1])
        h = jnp.dot(x, ws[v], preferred_element_type=_f32)
        g_ref[v] = dinv * h
        db_ref[v] = jnp.broadcast_to(dinv, h.shape)


def _tc_dense1(deg6, x, W1, fm1, fm2):
    nb = 10
    blk = NN // nb
    return pl.pallas_call(
        _dense1_body,
        grid=(nb,),
        in_specs=[
            pl.BlockSpec((6, blk, 16), lambda i: (0, i, 0)),
            pl.BlockSpec((blk, DD), lambda i: (i, 0)),
            pl.BlockSpec((DD, HH), lambda i: (0, 0)),
            pl.BlockSpec((1, DD), lambda i: (0, 0)),
            pl.BlockSpec((1, DD), lambda i: (0, 0)),
        ],
        out_specs=[
            pl.BlockSpec((3, blk, HH), lambda i: (0, i, 0)),
            pl.BlockSpec((3, blk, HH), lambda i: (0, i, 0)),
        ],
        out_shape=[jax.ShapeDtypeStruct((3, NN, HH), _f32)] * 2,
    )(deg6, x, W1, fm1, fm2)


def _spmm_body(g0_hbm, g1_hbm, g2_hbm, src_hbm, md0_hbm, md1_hbm, md2_hbm,
               out_hbm, srcv, md0, md1, md2, bA0, bA1, bA2, bB0, bB1, bB2,
               zbuf, acc, semgA, semgB, semsA, semsB):
    c = lax.axis_index("c")
    s = lax.axis_index("s")
    w = c * 16 + s

    zero16 = jnp.zeros((16,), _f32)

    def zinit(i, carry):
        zbuf[i, pl.ds(0, 16)] = zero16
        zbuf[i, pl.ds(16, 16)] = zero16
        return carry
    lax.fori_loop(0, ZR2, zinit, 0)

    for i in range(24):
        pltpu.sync_copy(zbuf, acc.at[pl.ds(s * STRIPE + i * ZR2, ZR2)])
    plsc.subcore_barrier()

    def pair(k, carry):
        jA = 2 * k
        jB = 2 * k + 1
        cA0 = pltpu.async_copy(g0_hbm.at[srcv.at[jA]], bA0, semgA)
        cA1 = pltpu.async_copy(g1_hbm.at[srcv.at[jA]], bA1, semgA)
        cA2 = pltpu.async_copy(g2_hbm.at[srcv.at[jA]], bA2, semgA)
        cB0 = pltpu.async_copy(g0_hbm.at[srcv.at[jB]], bB0, semgB)
        cB1 = pltpu.async_copy(g1_hbm.at[srcv.at[jB]], bB1, semgB)
        cB2 = pltpu.async_copy(g2_hbm.at[srcv.at[jB]], bB2, semgB)
        cA0.wait()
        cA1.wait()
        cA2.wait()
        sA0 = pltpu.async_copy(bA0, acc.at[md0.at[jA]], semsA, add=True)
        sA1 = pltpu.async_copy(bA1, acc.at[md1.at[jA]], semsA, add=True)
        sA2 = pltpu.async_copy(bA2, acc.at[md2.at[jA]], semsA, add=True)
        cB0.wait()
        cB1.wait()
        cB2.wait()
        sB0 = pltpu.async_copy(bB0, acc.at[md0.at[jB]], semsB, add=True)
        sB1 = pltpu.async_copy(bB1, acc.at[md1.at[jB]], semsB, add=True)
        sB2 = pltpu.async_copy(bB2, acc.at[md2.at[jB]], semsB, add=True)
        sA0.wait()
        sA1.wait()
        sA2.wait()
        sB0.wait()
        sB1.wait()
        sB2.wait()
        return carry

    for p in range(NPH):
        pltpu.sync_copy(src_hbm.at[w, pl.ds(p * RPP, RPP)], srcv)
        pltpu.sync_copy(md0_hbm.at[w, pl.ds(p * RPP, RPP)], md0)
        pltpu.sync_copy(md1_hbm.at[w, pl.ds(p * RPP, RPP)], md1)
        pltpu.sync_copy(md2_hbm.at[w, pl.ds(p * RPP, RPP)], md2)
        lax.fori_loop(0, RPP // 2, pair, 0)
    plsc.subcore_barrier()

    for v in range(3):
        pltpu.sync_copy(acc.at[pl.ds(v * NP2 + s * OUTW, OUTW)],
                        out_hbm.at[c, v, pl.ds(s * OUTW, OUTW)])


def _sc_spmm(g3, src_r, md0_r, md1_r, md2_r):
    mesh = plsc.VectorSubcoreMesh(core_axis_name="c", subcore_axis_name="s")
    return pl.kernel(
        _spmm_body,
        out_type=jax.ShapeDtypeStruct((2, 3, NP2, HH), _f32),
        mesh=mesh,
        compiler_params=pltpu.CompilerParams(use_tc_tiling_on_sc=False),
        scratch_types=[
            pltpu.VMEM((RPP, QW), _i32),
            pltpu.VMEM((RPP, QW), _i32),
            pltpu.VMEM((RPP, QW), _i32),
            pltpu.VMEM((RPP, QW), _i32),
            pltpu.VMEM((QW, HH), _f32),
            pltpu.VMEM((QW, HH), _f32),
            pltpu.VMEM((QW, HH), _f32),
            pltpu.VMEM((QW, HH), _f32),
            pltpu.VMEM((QW, HH), _f32),
            pltpu.VMEM((QW, HH), _f32),
            pltpu.VMEM((ZR2, HH), _f32),
            pltpu.VMEM_SHARED((ACC_ROWS, HH), _f32),
            pltpu.SemaphoreType.DMA,
            pltpu.SemaphoreType.DMA,
            pltpu.SemaphoreType.DMA,
            pltpu.SemaphoreType.DMA,
        ],
    )(g3[0], g3[1], g3[2], src_r, md0_r, md1_r, md2_r)


def _mid_body(acc_ref, g_ref, db_ref, b1_ref, w2_ref, g2_ref):
    w2 = w2_ref[...]
    b1 = b1_ref[...]
    for v in range(3):
        db = db_ref[v]
        t = db * (acc_ref[v] + acc_ref[v + 3] + g_ref[v]) + b1
        z = jnp.maximum(t, 0.0)
        g2_ref[v] = db * jnp.dot(z, w2, preferred_element_type=_f32)


def _tc_mid(acc6, G1, DB, b1, W2):
    nb = 10
    blk = NN // nb
    return pl.pallas_call(
        _mid_body,
        grid=(nb,),
        in_specs=[
            pl.BlockSpec((6, blk, HH), lambda i: (0, i, 0)),
            pl.BlockSpec((3, blk, HH), lambda i: (0, i, 0)),
            pl.BlockSpec((3, blk, HH), lambda i: (0, i, 0)),
            pl.BlockSpec((1, HH), lambda i: (0, 0)),
            pl.BlockSpec((HH, HH), lambda i: (0, 0)),
        ],
        out_specs=pl.BlockSpec((3, blk, HH), lambda i: (0, i, 0)),
        out_shape=jax.ShapeDtypeStruct((3, NN, HH), _f32),
    )(acc6, G1, DB, b1, W2)


def _fin_body(acc_ref, g_ref, db_ref, b2_ref, z_ref):
    b2 = b2_ref[...]
    for v in range(3):
        db = db_ref[v]
        t = db * (acc_ref[v] + acc_ref[v + 3] + g_ref[v]) + b2
        z_ref[v] = jnp.maximum(t, 0.0)


def _tc_fin(acc6, G2, DB, b2):
    nb = 10
    blk = NN // nb
    return pl.pallas_call(
        _fin_body,
        grid=(nb,),
        in_specs=[
            pl.BlockSpec((6, blk, HH), lambda i: (0, i, 0)),
            pl.BlockSpec((3, blk, HH), lambda i: (0, i, 0)),
            pl.BlockSpec((3, blk, HH), lambda i: (0, i, 0)),
            pl.BlockSpec((1, HH), lambda i: (0, 0)),
        ],
        out_specs=pl.BlockSpec((3, blk, HH), lambda i: (0, i, 0)),
        out_shape=jax.ShapeDtypeStruct((3, NN, HH), _f32),
    )(acc6, G2, DB, b2)


def kernel(x, edge_index, W1, b1, W2, b2,
           edge_mask1, feat_mask1, edge_mask2, feat_mask2):
    src = edge_index[0]
    dst = edge_index[1]
    npad = EPAD - EE
    src_p = jnp.concatenate([src, jnp.zeros((npad,), _i32)])
    dst_p = jnp.concatenate([dst, jnp.full((npad,), NN, _i32)])
    m1_p = jnp.concatenate([edge_mask1.astype(_i32), jnp.zeros((npad,), _i32)])
    m2_p = jnp.concatenate([edge_mask2.astype(_i32), jnp.zeros((npad,), _i32)])

    mdst1, mdst2 = _tc_mdst(dst_p.reshape(NW, 1, EPW),
                            m1_p.reshape(NW, 1, EPW),
                            m2_p.reshape(NW, 1, EPW))

    src_r = src_p.reshape(NW, NR, QW)
    md0_r = dst_p.reshape(NW, NR, QW)
    md1_r = mdst1.reshape(NW, NR, QW)
    md2_r = mdst2.reshape(NW, NR, QW)

    deg = _sc_deg(md0_r, md1_r, md2_r)
    deg6 = deg[:, :, :NN, :].reshape(6, NN, 16)

    fm1 = feat_mask1.astype(_f32).reshape(1, DD)
    fm2 = feat_mask2.astype(_f32).reshape(1, DD)
    G1, DB = _tc_dense1(deg6, x, W1, fm1, fm2)

    acc1 = _sc_spmm(G1, src_r, md0_r, md1_r, md2_r)[:, :, :NN, :].reshape(6, NN, HH)
    G2 = _tc_mid(acc1, G1, DB, b1.reshape(1, HH), W2)
    acc2 = _sc_spmm(G2, src_r, md0_r, md1_r, md2_r)[:, :, :NN, :].reshape(6, NN, HH)
    Z = _tc_fin(acc2, G2, DB, b2.reshape(1, HH))
    return (Z[0], Z[1], Z[2])
